# speedup vs baseline: 30.9174x; 30.9174x over previous
import jax
import jax.numpy as jnp
from jax.experimental import pallas as pl

H = 16
HD = 64
VD = 64
SCALE = HD ** -0.5


def _proj_kernel(x_ref, w_ref, o_ref):
    o_ref[...] = jax.lax.dot_general(
        x_ref[...], w_ref[...], (((1,), (0,)), ((), ())),
        preferred_element_type=jnp.float32)


def _head_kernel(q_ref, k_ref, v_ref, o1_ref, o2_ref):
    q = q_ref[0]
    k = k_ref[0]
    v = v_ref[0]
    n = q.shape[0]

    p = jax.lax.dot_general(q, k, (((1,), (1,)), ((), ())),
                            preferred_element_type=jnp.float32)

    m1 = jnp.max(p, axis=1, keepdims=True)
    e1 = jnp.exp((p - m1) * SCALE)
    att1 = e1 / jnp.sum(e1, axis=1, keepdims=True)
    o1_ref[0] = jax.lax.dot_general(att1, v, (((1,), (0,)), ((), ())),
                                    preferred_element_type=jnp.float32)

    iota = jax.lax.broadcasted_iota(jnp.int32, p.shape, 1)
    idx = jnp.min(jnp.where(p == m1, iota, 2 * n), axis=1, keepdims=True)
    valid = m1 > 0.0
    sel = jnp.where((iota == idx) & valid, 1.0, 0.0)
    km = jax.lax.dot_general(sel, k, (((1,), (0,)), ((), ())),
                             preferred_element_type=jnp.float32)
    vm = jax.lax.dot_general(sel, v, (((1,), (0,)), ((), ())),
                             preferred_element_type=jnp.float32)

    p2 = jax.lax.dot_general(q, km, (((1,), (1,)), ((), ())),
                             preferred_element_type=jnp.float32)
    m2 = jnp.max(p2, axis=1, keepdims=True)
    e2 = jnp.exp((p2 - m2) * SCALE)
    att2 = e2 / jnp.sum(e2, axis=1, keepdims=True)
    o2_ref[0] = jax.lax.dot_general(att2, vm, (((1,), (0,)), ((), ())),
                                    preferred_element_type=jnp.float32)


def _combine_kernel(o1_ref, o2_ref, w_ref, wout_ref, bout_ref, out_ref):
    a1 = jax.lax.dot_general(o1_ref[...], wout_ref[...], (((1,), (0,)), ((), ())),
                             preferred_element_type=jnp.float32) + bout_ref[...]
    a2 = jax.lax.dot_general(o2_ref[...], wout_ref[...], (((1,), (0,)), ((), ())),
                             preferred_element_type=jnp.float32) + bout_ref[...]
    g = jax.nn.sigmoid(w_ref[...])
    out_ref[...] = g * a1 + (1.0 - g) * a2


@jax.jit
def kernel(x, Wq, Wk, Wv, Ww, Wout, bout):
    b, n, d_in = x.shape
    d_out = Wout.shape[0]
    x2 = x.reshape(n, d_in)

    wcat = jnp.concatenate([Wq, Wk, Wv, Ww], axis=0).T
    nt = 8
    qkvw = pl.pallas_call(
        _proj_kernel,
        grid=(nt,),
        in_specs=[
            pl.BlockSpec((n // nt, d_in), lambda i: (i, 0)),
            pl.BlockSpec((d_in, 4 * d_in), lambda i: (0, 0)),
        ],
        out_specs=pl.BlockSpec((n // nt, 4 * d_in), lambda i: (i, 0)),
        out_shape=jax.ShapeDtypeStruct((n, 4 * d_in), jnp.float32),
    )(x2, wcat)
    q, k, v, w = (qkvw[:, 0:d_in], qkvw[:, d_in:2 * d_in],
                  qkvw[:, 2 * d_in:3 * d_in], qkvw[:, 3 * d_in:4 * d_in])

    qh = q.reshape(n, H, HD).transpose(1, 0, 2)
    kh = k.reshape(n, H, HD).transpose(1, 0, 2)
    vh = v.reshape(n, H, VD).transpose(1, 0, 2)

    o1h, o2h = pl.pallas_call(
        _head_kernel,
        grid=(H,),
        in_specs=[
            pl.BlockSpec((1, n, HD), lambda h: (h, 0, 0)),
            pl.BlockSpec((1, n, HD), lambda h: (h, 0, 0)),
            pl.BlockSpec((1, n, VD), lambda h: (h, 0, 0)),
        ],
        out_specs=[
            pl.BlockSpec((1, n, VD), lambda h: (h, 0, 0)),
            pl.BlockSpec((1, n, VD), lambda h: (h, 0, 0)),
        ],
        out_shape=[
            jax.ShapeDtypeStruct((H, n, VD), jnp.float32),
            jax.ShapeDtypeStruct((H, n, VD), jnp.float32),
        ],
    )(qh, kh, vh)

    o1 = o1h.transpose(1, 0, 2).reshape(n, H * VD)
    o2 = o2h.transpose(1, 0, 2).reshape(n, H * VD)

    out = pl.pallas_call(
        _combine_kernel,
        grid=(nt,),
        in_specs=[
            pl.BlockSpec((n // nt, d_out), lambda i: (i, 0)),
            pl.BlockSpec((n // nt, d_out), lambda i: (i, 0)),
            pl.BlockSpec((n // nt, d_out), lambda i: (i, 0)),
            pl.BlockSpec((d_out, d_out), lambda i: (0, 0)),
            pl.BlockSpec((d_out,), lambda i: (0,)),
        ],
        out_specs=pl.BlockSpec((n // nt, d_out), lambda i: (i, 0)),
        out_shape=jax.ShapeDtypeStruct((n, d_out), jnp.float32),
    )(o1, o2, w, Wout.T, bout)

    return out.reshape(b, n, d_out)


# speedup vs baseline: 34.1039x; 1.1031x over previous
import jax
import jax.numpy as jnp
from jax.experimental import pallas as pl

H = 16
HD = 64
VD = 64
SCALE = HD ** -0.5


def _dot(a, b, dims):
    return jax.lax.dot_general(a, b, (dims, ((), ())),
                               preferred_element_type=jnp.float32)


def _proj_kernel(x_ref, w_ref, o_ref):
    o_ref[...] = _dot(x_ref[...], w_ref[...], ((1,), (1,)))


def _head_kernel(q_ref, k_ref, v_ref, o1_ref, o2_ref):
    q = q_ref[0]
    k = k_ref[0]
    v = v_ref[0]
    n = q.shape[0]

    p = _dot(q, k, ((1,), (1,)))

    m1 = jnp.max(p, axis=1, keepdims=True)
    e1 = jnp.exp((p - m1) * SCALE)
    s1 = jnp.sum(e1, axis=1, keepdims=True)
    o1_ref[0] = _dot(e1, v, ((1,), (0,))) / s1

    iota = jax.lax.broadcasted_iota(jnp.int32, p.shape, 1)
    idx = jnp.min(jnp.where(p == m1, iota, 2 * n), axis=1, keepdims=True)
    valid = m1 > 0.0
    sel = jnp.where((iota == idx) & valid, 1.0, 0.0)
    km = _dot(sel, k, ((1,), (0,)))
    vm = _dot(sel, v, ((1,), (0,)))

    p2 = _dot(q, km, ((1,), (1,)))
    m2 = jnp.max(p2, axis=1, keepdims=True)
    e2 = jnp.exp((p2 - m2) * SCALE)
    s2 = jnp.sum(e2, axis=1, keepdims=True)
    o2_ref[0] = _dot(e2, vm, ((1,), (0,))) / s2


def _combine_kernel(o1_ref, o2_ref, w_ref, wout_ref, bout_ref, out_ref):
    a1 = _dot(o1_ref[...], wout_ref[...], ((1,), (1,)))
    a2 = _dot(o2_ref[...], wout_ref[...], ((1,), (1,)))
    g = jax.nn.sigmoid(w_ref[...])
    out_ref[...] = g * (a1 - a2) + a2 + bout_ref[...]


@jax.jit
def kernel(x, Wq, Wk, Wv, Ww, Wout, bout):
    b, n, d_in = x.shape
    d_out = Wout.shape[0]
    x2 = x.reshape(n, d_in)

    wcat = jnp.concatenate([Wq, Wk, Wv, Ww], axis=0)
    nt = 8
    qkvw = pl.pallas_call(
        _proj_kernel,
        grid=(nt,),
        in_specs=[
            pl.BlockSpec((n // nt, d_in), lambda i: (i, 0)),
            pl.BlockSpec((4 * d_in, d_in), lambda i: (0, 0)),
        ],
        out_specs=pl.BlockSpec((n // nt, 4 * d_in), lambda i: (i, 0)),
        out_shape=jax.ShapeDtypeStruct((n, 4 * d_in), jnp.float32),
    )(x2, wcat)

    q = jax.lax.slice(qkvw, (0, 0), (n, d_in))
    k = jax.lax.slice(qkvw, (0, d_in), (n, 2 * d_in))
    v = jax.lax.slice(qkvw, (0, 2 * d_in), (n, 3 * d_in))
    w_gate = jax.lax.slice(qkvw, (0, 3 * d_in), (n, 4 * d_in))
    qh = q.reshape(n, H, HD).transpose(1, 0, 2)
    kh = k.reshape(n, H, HD).transpose(1, 0, 2)
    vh = v.reshape(n, H, VD).transpose(1, 0, 2)

    o1h, o2h = pl.pallas_call(
        _head_kernel,
        grid=(H,),
        in_specs=[
            pl.BlockSpec((1, n, HD), lambda h: (h, 0, 0)),
            pl.BlockSpec((1, n, HD), lambda h: (h, 0, 0)),
            pl.BlockSpec((1, n, VD), lambda h: (h, 0, 0)),
        ],
        out_specs=[
            pl.BlockSpec((1, n, VD), lambda h: (h, 0, 0)),
            pl.BlockSpec((1, n, VD), lambda h: (h, 0, 0)),
        ],
        out_shape=[
            jax.ShapeDtypeStruct((H, n, VD), jnp.float32),
            jax.ShapeDtypeStruct((H, n, VD), jnp.float32),
        ],
    )(qh, kh, vh)

    o1 = o1h.transpose(1, 0, 2).reshape(n, H * VD)
    o2 = o2h.transpose(1, 0, 2).reshape(n, H * VD)

    out = pl.pallas_call(
        _combine_kernel,
        grid=(nt,),
        in_specs=[
            pl.BlockSpec((n // nt, d_out), lambda i: (i, 0)),
            pl.BlockSpec((n // nt, d_out), lambda i: (i, 0)),
            pl.BlockSpec((n // nt, d_out), lambda i: (i, 0)),
            pl.BlockSpec((d_out, d_out), lambda i: (0, 0)),
            pl.BlockSpec((d_out,), lambda i: (0,)),
        ],
        out_specs=pl.BlockSpec((n // nt, d_out), lambda i: (i, 0)),
        out_shape=jax.ShapeDtypeStruct((n, d_out), jnp.float32),
    )(o1, o2, w_gate, Wout, bout)

    return out.reshape(b, n, d_out)


# speedup vs baseline: 40.5799x; 1.1899x over previous
import jax
import jax.numpy as jnp
from jax.experimental import pallas as pl

H = 16
HD = 64
VD = 64
SCALE = HD ** -0.5


def _dot(a, b, dims):
    return jax.lax.dot_general(a, b, (dims, ((), ())),
                               preferred_element_type=jnp.float32)


def _proj_kernel(x_ref, w_ref, qkv_ref, wg_ref):
    o = _dot(x_ref[...], w_ref[...], ((1,), (1,)))
    for g in range(3 * H):
        qkv_ref[g] = o[:, g * HD:(g + 1) * HD]
    wg_ref[...] = o[:, 3 * H * HD:]


def _head_kernel(q_ref, k_ref, v_ref, o1_ref, o2_ref):
    q = q_ref[0]
    k = k_ref[0]
    v = v_ref[0]
    n = q.shape[0]

    p = _dot(q, k, ((1,), (1,)))

    m1 = jnp.max(p, axis=1, keepdims=True)
    e1 = jnp.exp((p - m1) * SCALE)
    s1 = jnp.sum(e1, axis=1, keepdims=True)
    o1_ref[0] = _dot(e1, v, ((1,), (0,))) / s1

    iota = jax.lax.broadcasted_iota(jnp.int32, p.shape, 1)
    idx = jnp.min(jnp.where(p == m1, iota, 2 * n), axis=1, keepdims=True)
    valid = m1 > 0.0
    sel = jnp.where((iota == idx) & valid, 1.0, 0.0)
    km = _dot(sel, k, ((1,), (0,)))
    vm = _dot(sel, v, ((1,), (0,)))

    p2 = _dot(q, km, ((1,), (1,)))
    m2 = jnp.max(p2, axis=1, keepdims=True)
    e2 = jnp.exp((p2 - m2) * SCALE)
    s2 = jnp.sum(e2, axis=1, keepdims=True)
    o2_ref[0] = _dot(e2, vm, ((1,), (0,))) / s2


def _combine_kernel(o1_ref, o2_ref, wg_ref, wout_ref, bout_ref, out_ref):
    a1 = _dot(o1_ref[0], wout_ref[0], ((1,), (0,)))
    a2 = _dot(o2_ref[0], wout_ref[0], ((1,), (0,)))
    for h in range(1, H):
        a1 = a1 + _dot(o1_ref[h], wout_ref[h], ((1,), (0,)))
        a2 = a2 + _dot(o2_ref[h], wout_ref[h], ((1,), (0,)))
    g = jax.nn.sigmoid(wg_ref[...])
    out_ref[...] = g * (a1 - a2) + a2 + bout_ref[...]


@jax.jit
def kernel(x, Wq, Wk, Wv, Ww, Wout, bout):
    b, n, d_in = x.shape
    d_out = Wout.shape[0]
    x2 = x.reshape(n, d_in)

    wcat = jnp.concatenate([Wq, Wk, Wv, Ww], axis=0)
    nt = 8
    qkv3, wg = pl.pallas_call(
        _proj_kernel,
        grid=(nt,),
        in_specs=[
            pl.BlockSpec((n // nt, d_in), lambda i: (i, 0)),
            pl.BlockSpec((4 * d_in, d_in), lambda i: (0, 0)),
        ],
        out_specs=[
            pl.BlockSpec((3 * H, n // nt, HD), lambda i: (0, i, 0)),
            pl.BlockSpec((n // nt, d_in), lambda i: (i, 0)),
        ],
        out_shape=[
            jax.ShapeDtypeStruct((3 * H, n, HD), jnp.float32),
            jax.ShapeDtypeStruct((n, d_in), jnp.float32),
        ],
    )(x2, wcat)

    o1h, o2h = pl.pallas_call(
        _head_kernel,
        grid=(H,),
        in_specs=[
            pl.BlockSpec((1, n, HD), lambda h: (h, 0, 0)),
            pl.BlockSpec((1, n, HD), lambda h: (H + h, 0, 0)),
            pl.BlockSpec((1, n, VD), lambda h: (2 * H + h, 0, 0)),
        ],
        out_specs=[
            pl.BlockSpec((1, n, VD), lambda h: (h, 0, 0)),
            pl.BlockSpec((1, n, VD), lambda h: (h, 0, 0)),
        ],
        out_shape=[
            jax.ShapeDtypeStruct((H, n, VD), jnp.float32),
            jax.ShapeDtypeStruct((H, n, VD), jnp.float32),
        ],
    )(qkv3, qkv3, qkv3)

    wout3 = Wout.T.reshape(H, VD, d_out)
    out = pl.pallas_call(
        _combine_kernel,
        grid=(nt,),
        in_specs=[
            pl.BlockSpec((H, n // nt, VD), lambda i: (0, i, 0)),
            pl.BlockSpec((H, n // nt, VD), lambda i: (0, i, 0)),
            pl.BlockSpec((n // nt, d_out), lambda i: (i, 0)),
            pl.BlockSpec((H, VD, d_out), lambda i: (0, 0, 0)),
            pl.BlockSpec((d_out,), lambda i: (0,)),
        ],
        out_specs=pl.BlockSpec((n // nt, d_out), lambda i: (i, 0)),
        out_shape=jax.ShapeDtypeStruct((n, d_out), jnp.float32),
    )(o1h, o2h, wg, wout3, bout)

    return out.reshape(b, n, d_out)
